# baseline (speedup 1.0000x reference)
import functools

import jax
import jax.numpy as jnp
from jax.experimental import pallas as pl
from jax.experimental.pallas import tpu as pltpu

NTOK = 2048
H = 2048
DFF = 5632
NSLOT = 4

MT = 256
FT = 512
NF = DFF // FT
NM = NTOK // MT
AFT = 256
NAF = 512 // AFT


def _silu(g):
    return g * jax.nn.sigmoid(g)


def _dot_t(a, b):
    return jax.lax.dot_general(
        a.astype(jnp.bfloat16), b.astype(jnp.bfloat16),
        (((1,), (1,)), ((), ())), preferred_element_type=jnp.float32)


def _base_body(x_ref, gw_ref, uw_ref, dw_ref, out_ref):
    f = pl.program_id(0)
    m = pl.program_id(1)
    xm = x_ref[pl.ds(m * MT, MT), :]
    g = _dot_t(xm, gw_ref[...])
    u = _dot_t(xm, uw_ref[...])
    h = _silu(g) * u
    contrib = jax.lax.dot_general(
        h.astype(jnp.bfloat16), dw_ref[...].astype(jnp.bfloat16),
        (((1,), (1,)), ((), ())), preferred_element_type=jnp.float32)

    @pl.when(f == 0)
    def _():
        out_ref[pl.ds(m * MT, MT), :] = contrib

    @pl.when(f != 0)
    def _():
        out_ref[pl.ds(m * MT, MT), :] += contrib


def _adapter_body(ti_ref, scales_ref, x_ref, base_ref,
                  rg_ref, ru_ref, rd_ref, fg_ref, fu_ref, fd_ref, out_ref):
    m = pl.program_id(0)
    s = pl.program_id(1)
    xm = x_ref[...]
    mask = (ti_ref[pl.ds(m * MT, MT)] == s).astype(jnp.float32)[:, None]
    rs = scales_ref[s, 0]
    fs = scales_ref[s, 1]

    gr = _dot_t(xm, rg_ref[0])
    ur = _dot_t(xm, ru_ref[0])
    hr = _silu(gr) * ur * (mask * rs)
    contrib = jax.lax.dot_general(
        hr.astype(jnp.bfloat16), rd_ref[0].astype(jnp.bfloat16),
        (((1,), (1,)), ((), ())), preferred_element_type=jnp.float32)

    gf = _dot_t(xm, fg_ref[0])
    uf = _dot_t(xm, fu_ref[0])
    hf = _silu(gf) * uf * (mask * fs)
    contrib += jax.lax.dot_general(
        hf.astype(jnp.bfloat16), fd_ref[0].astype(jnp.bfloat16),
        (((1,), (1,)), ((), ())), preferred_element_type=jnp.float32)

    first = (s == 0) & (pl.program_id(2) == 0)

    @pl.when(first)
    def _():
        out_ref[...] = base_ref[...] + contrib

    @pl.when(jnp.logical_not(first))
    def _():
        out_ref[...] += contrib


def kernel(x, token_indices, gate_w, up_w, down_w, retain_gate, retain_up,
           retain_down, forget_gate, forget_up, forget_down, scales):
    full = pl.BlockSpec((NTOK, H), lambda *_: (0, 0))

    base_out = pl.pallas_call(
        _base_body,
        grid=(NF, NM),
        in_specs=[
            full,
            pl.BlockSpec((FT, H), lambda f, m: (f, 0)),
            pl.BlockSpec((FT, H), lambda f, m: (f, 0)),
            pl.BlockSpec((H, FT), lambda f, m: (0, f)),
        ],
        out_specs=full,
        out_shape=jax.ShapeDtypeStruct((NTOK, H), jnp.float32),
        compiler_params=pltpu.CompilerParams(
            dimension_semantics=("arbitrary", "arbitrary")),
    )(x, gate_w, up_w, down_w)

    ti = token_indices.astype(jnp.int32)

    out = pl.pallas_call(
        _adapter_body,
        grid=(NM, NSLOT, NAF),
        in_specs=[
            pl.BlockSpec((NTOK,), lambda m, s, f: (0,)),
            pl.BlockSpec(memory_space=pltpu.SMEM),
            pl.BlockSpec((MT, H), lambda m, s, f: (m, 0)),
            pl.BlockSpec((MT, H), lambda m, s, f: (m, 0)),
            pl.BlockSpec((1, AFT, H), lambda m, s, f: (s, f, 0)),
            pl.BlockSpec((1, AFT, H), lambda m, s, f: (s, f, 0)),
            pl.BlockSpec((1, H, AFT), lambda m, s, f: (s, 0, f)),
            pl.BlockSpec((1, AFT, H), lambda m, s, f: (s, f, 0)),
            pl.BlockSpec((1, AFT, H), lambda m, s, f: (s, f, 0)),
            pl.BlockSpec((1, H, AFT), lambda m, s, f: (s, 0, f)),
        ],
        out_specs=pl.BlockSpec((MT, H), lambda m, s, f: (m, 0)),
        out_shape=jax.ShapeDtypeStruct((NTOK, H), jnp.float32),
        compiler_params=pltpu.CompilerParams(
            dimension_semantics=("arbitrary", "arbitrary", "arbitrary")),
    )(ti, scales, x, base_out,
      retain_gate, retain_up, retain_down,
      forget_gate, forget_up, forget_down)

    return out
